# baseline (speedup 1.0000x reference)
import jax
import jax.numpy as jnp
from jax.experimental import pallas as pl
from jax.experimental.pallas import tpu as pltpu

_OUT_DOM = 64
_DIM_CONT = 128


def _body(ids_ref, x_ref, dvec_ref, w_ref, b_ref, t_hbm, out_ref, rowbuf, sem):
    cin = x_ref.shape[1]
    hc = x_ref.shape[2]
    wd = x_ref.shape[3]
    i = pl.program_id(0)
    j = pl.program_id(1)

    @pl.when(j == 0)
    def _fetch_row():
        pltpu.make_async_copy(
            t_hbm.at[pl.ds(ids_ref[i], 1)], rowbuf, sem).start()

    out_ref[0, :cin] = x_ref[0]

    @pl.when(j == 0)
    def _wait_row():
        pltpu.make_async_copy(
            t_hbm.at[pl.ds(ids_ref[i], 1)], rowbuf, sem).wait()

    dvv = dvec_ref[0]
    dv = jnp.maximum(
        jnp.dot(dvv, w_ref[...], preferred_element_type=jnp.float32) + b_ref[...],
        0.0,
    )
    dv = dv + rowbuf[...]
    out_ref[0, cin:] = jnp.broadcast_to(
        dv.reshape(_OUT_DOM, 1, 1), (_OUT_DOM, hc, wd))


def kernel(x, domain_ids, domain_vectors, W, b, table):
    bsz, cin, h, w = x.shape
    cout = cin + _OUT_DOM
    hc = 32
    nh = h // hc

    dvec3 = domain_vectors.reshape(bsz, 1, _DIM_CONT)
    b2 = b.reshape(1, _OUT_DOM)

    return pl.pallas_call(
        _body,
        grid_spec=pltpu.PrefetchScalarGridSpec(
            num_scalar_prefetch=1,
            grid=(bsz, nh),
            in_specs=[
                pl.BlockSpec((1, cin, hc, w), lambda i, j, ids: (i, 0, j, 0)),
                pl.BlockSpec((1, 1, _DIM_CONT), lambda i, j, ids: (i, 0, 0)),
                pl.BlockSpec((_DIM_CONT, _OUT_DOM), lambda i, j, ids: (0, 0)),
                pl.BlockSpec((1, _OUT_DOM), lambda i, j, ids: (0, 0)),
                pl.BlockSpec(memory_space=pltpu.HBM),
            ],
            out_specs=pl.BlockSpec((1, cout, hc, w), lambda i, j, ids: (i, 0, j, 0)),
            scratch_shapes=[
                pltpu.VMEM((1, _OUT_DOM), jnp.float32),
                pltpu.SemaphoreType.DMA,
            ],
        ),
        out_shape=jax.ShapeDtypeStruct((bsz, cout, h, w), x.dtype),
    )(domain_ids, x, dvec3, W, b2, table)


# speedup vs baseline: 1.0031x; 1.0031x over previous
import jax
import jax.numpy as jnp
from jax.experimental import pallas as pl
from jax.experimental.pallas import tpu as pltpu

_OUT_DOM = 64
_DIM_CONT = 128


def _dv_body(ids_ref, dvec_ref, w_ref, b_ref, t_hbm, dv_ref, rowbuf, sem):
    bsz = dvec_ref.shape[0]
    cps = []
    for i in range(bsz):
        cp = pltpu.make_async_copy(
            t_hbm.at[pl.ds(ids_ref[i], 1)], rowbuf.at[pl.ds(i, 1)], sem.at[i])
        cp.start()
        cps.append(cp)
    for cp in cps:
        cp.wait()
    dv_ref[...] = jnp.maximum(
        jnp.dot(dvec_ref[...], w_ref[...], preferred_element_type=jnp.float32)
        + b_ref[...],
        0.0,
    ) + rowbuf[...]


def _concat_body(x_ref, dv_ref, out_ref):
    cin = x_ref.shape[1]
    hc = x_ref.shape[2]
    wd = x_ref.shape[3]
    out_ref[0, :cin] = x_ref[0]
    dv = dv_ref[0]
    out_ref[0, cin:] = jnp.broadcast_to(
        dv.reshape(_OUT_DOM, 1, 1), (_OUT_DOM, hc, wd))


def kernel(x, domain_ids, domain_vectors, W, b, table):
    bsz, cin, h, w = x.shape
    cout = cin + _OUT_DOM
    hc = 32
    nh = h // hc

    b2 = b.reshape(1, _OUT_DOM)

    dv = pl.pallas_call(
        _dv_body,
        in_specs=[
            pl.BlockSpec(memory_space=pltpu.SMEM),
            pl.BlockSpec(memory_space=pltpu.VMEM),
            pl.BlockSpec(memory_space=pltpu.VMEM),
            pl.BlockSpec(memory_space=pltpu.VMEM),
            pl.BlockSpec(memory_space=pltpu.HBM),
        ],
        out_specs=pl.BlockSpec(memory_space=pltpu.VMEM),
        out_shape=jax.ShapeDtypeStruct((bsz, _OUT_DOM), jnp.float32),
        scratch_shapes=[
            pltpu.VMEM((bsz, _OUT_DOM), jnp.float32),
            pltpu.SemaphoreType.DMA((bsz,)),
        ],
    )(domain_ids, domain_vectors, W, b2, table)

    dv3 = dv.reshape(bsz, 1, _OUT_DOM)

    return pl.pallas_call(
        _concat_body,
        grid=(bsz, nh),
        in_specs=[
            pl.BlockSpec((1, cin, hc, w), lambda i, j: (i, 0, j, 0)),
            pl.BlockSpec((1, 1, _OUT_DOM), lambda i, j: (i, 0, 0)),
        ],
        out_specs=pl.BlockSpec((1, cout, hc, w), lambda i, j: (i, 0, j, 0)),
        out_shape=jax.ShapeDtypeStruct((bsz, cout, h, w), x.dtype),
    )(x, dv3)


# speedup vs baseline: 1.0091x; 1.0060x over previous
import jax
import jax.numpy as jnp
from jax.experimental import pallas as pl
from jax.experimental.pallas import tpu as pltpu

_OUT_DOM = 64
_DIM_CONT = 128


def _body(ids_ref, x_ref, dvec_ref, w_ref, b_ref, ttile_ref, out_ref):
    cin = x_ref.shape[1]
    hc = x_ref.shape[2]
    wd = x_ref.shape[3]
    i = pl.program_id(0)
    row = ids_ref[i] % 8
    trow = ttile_ref[pl.ds(row, 1), :]
    dvv = dvec_ref[0]
    dv = jnp.maximum(
        jnp.dot(dvv, w_ref[...], preferred_element_type=jnp.float32) + b_ref[...],
        0.0,
    ) + trow
    out_ref[0, :cin] = x_ref[0]
    out_ref[0, cin:] = jnp.broadcast_to(
        dv.reshape(_OUT_DOM, 1, 1), (_OUT_DOM, hc, wd))


def kernel(x, domain_ids, domain_vectors, W, b, table):
    bsz, cin, h, w = x.shape
    cout = cin + _OUT_DOM
    hc = 32
    nh = h // hc

    dvec3 = domain_vectors.reshape(bsz, 1, _DIM_CONT)
    b2 = b.reshape(1, _OUT_DOM)

    return pl.pallas_call(
        _body,
        grid_spec=pltpu.PrefetchScalarGridSpec(
            num_scalar_prefetch=1,
            grid=(bsz, nh),
            in_specs=[
                pl.BlockSpec((1, cin, hc, w), lambda i, j, ids: (i, 0, j, 0)),
                pl.BlockSpec((1, 1, _DIM_CONT), lambda i, j, ids: (i, 0, 0)),
                pl.BlockSpec((_DIM_CONT, _OUT_DOM), lambda i, j, ids: (0, 0)),
                pl.BlockSpec((1, _OUT_DOM), lambda i, j, ids: (0, 0)),
                pl.BlockSpec((8, _OUT_DOM), lambda i, j, ids: (ids[i] // 8, 0)),
            ],
            out_specs=pl.BlockSpec((1, cout, hc, w), lambda i, j, ids: (i, 0, j, 0)),
        ),
        out_shape=jax.ShapeDtypeStruct((bsz, cout, h, w), x.dtype),
    )(domain_ids, x, dvec3, W, b2, table)
